# baseline (speedup 1.0000x reference)
import jax
import jax.numpy as jnp
from jax import lax
from jax.experimental import pallas as pl
from jax.experimental.pallas import tpu as pltpu
from jax.experimental.pallas import tpu_sc as plsc

BATCH = 4
SEQ = 2048
D = 64
DP = 128
VOCAB = 100000
NTOK = BATCH * SEQ

NC = 2
NS = 16
L = 16
NW = NC * NS
B_W = NTOK // NW
CHUNK = 128
NCH = B_W // CHUNK
SEQ_W = SEQ // B_W

VBLK = 128
NFULL = VOCAB // VBLK
VREM = VOCAB - NFULL * VBLK
BLK_PER_W = -(-NFULL // NW)

_PARAMS = pltpu.CompilerParams(use_tc_tiling_on_sc=True, needs_layout_passes=False)
_MESH = plsc.VectorSubcoreMesh(core_axis_name="c", subcore_axis_name="s")


def _wid():
    return lax.axis_index("s") * NC + lax.axis_index("c")


def _transpose_block(blk_v, row_v, nv):
    def _tok(v, carry):
        vvec = jnp.full((L,), v, dtype=jnp.int32)
        for c in range(D // L):
            dvec = lax.iota(jnp.int32, L) + (c * L)
            row_v[v, pl.ds(c * L, L)] = plsc.load_gather(blk_v, [dvec, vvec])
        return carry

    lax.fori_loop(0, nv, _tok, 0)


def _relayout_body(tokT_hbm, tail_hbm, rows_hbm, blk_v, row_v, sem):
    wid = _wid()

    @pl.when(wid == NW - 1)
    def _tail():
        pltpu.sync_copy(tail_hbm, row_v.at[pl.ds(0, VREM)])
        pltpu.sync_copy(row_v.at[pl.ds(0, VREM)], rows_hbm.at[pl.ds(NFULL * VBLK, VREM)])

    def _blk(i, carry):
        j = wid + i * NW

        @pl.when(j < NFULL)
        def _full():
            pltpu.sync_copy(tokT_hbm.at[:, pl.ds(j * VBLK, VBLK)], blk_v)
            _transpose_block(blk_v, row_v, VBLK)
            pltpu.sync_copy(row_v, rows_hbm.at[pl.ds(j * VBLK, VBLK)])

        return carry

    lax.fori_loop(0, BLK_PER_W, _blk, 0)


def _gather_body(x_hbm, rows_hbm, pos_hbm, out_hbm, idx_v, rows_v, pos_v, out_v, sem):
    wid = _wid()
    b = wid // SEQ_W
    s0 = (wid % SEQ_W) * B_W
    pltpu.sync_copy(x_hbm.at[b, pl.ds(s0, B_W)], idx_v)
    copies = [
        pltpu.async_copy(
            rows_hbm.at[idx_v.at[pl.ds(c * CHUNK, CHUNK)]],
            rows_v.at[pl.ds(c * CHUNK, CHUNK)],
            sem,
        )
        for c in range(NCH)
    ]
    pltpu.sync_copy(pos_hbm.at[:, pl.ds(s0, B_W)], pos_v)
    for cp in copies:
        cp.wait()

    def _dim(d, carry):
        dvec = jnp.full((L,), d, dtype=jnp.int32)
        for g in range(B_W // L):
            tvec = lax.iota(jnp.int32, L) + (g * L)
            val = plsc.load_gather(rows_v, [tvec, dvec])
            out_v[d, pl.ds(g * L, L)] = val + pos_v[d, pl.ds(g * L, L)]
        return carry

    lax.fori_loop(0, D, _dim, 0)
    pltpu.sync_copy(out_v, out_hbm.at[b, :, pl.ds(s0, B_W)])


def kernel(x, token_table, pos_table):
    tokT = token_table.T
    posT = pos_table.T
    tail_pad = jnp.pad(token_table[NFULL * VBLK :, :], ((0, 0), (0, DP - D)))
    rows = pl.kernel(
        _relayout_body,
        mesh=_MESH,
        out_type=jax.ShapeDtypeStruct((VOCAB, DP), jnp.float32),
        scratch_types=[
            pltpu.VMEM((D, VBLK), jnp.float32),
            pltpu.VMEM((VBLK, DP), jnp.float32),
            pltpu.SemaphoreType.DMA,
        ],
        compiler_params=_PARAMS,
    )(tokT, tail_pad)
    outT = pl.kernel(
        _gather_body,
        mesh=_MESH,
        out_type=jax.ShapeDtypeStruct((BATCH, D, SEQ), jnp.float32),
        scratch_types=[
            pltpu.VMEM((B_W,), jnp.int32),
            pltpu.VMEM((B_W, DP), jnp.float32),
            pltpu.VMEM((D, B_W), jnp.float32),
            pltpu.VMEM((D, B_W), jnp.float32),
            pltpu.SemaphoreType.DMA,
        ],
        compiler_params=_PARAMS,
    )(x, rows, posT)
    return outT.transpose(0, 2, 1)


# speedup vs baseline: 1.3175x; 1.3175x over previous
import jax
import jax.numpy as jnp
from jax import lax
from jax.experimental import pallas as pl
from jax.experimental.pallas import tpu as pltpu
from jax.experimental.pallas import tpu_sc as plsc

BATCH = 4
SEQ = 2048
D = 64
DP = 128
VOCAB = 100000
NTOK = BATCH * SEQ

NC = 2
NS = 16
L = 16
NW = NC * NS
B_W = NTOK // NW
SEQ_W = SEQ // B_W

VBLK = 128
NFULL = VOCAB // VBLK
VREM = VOCAB - NFULL * VBLK
NBLK = NFULL + 1
BLK_PER_W = -(-NBLK // NW)
NCHUNK = NTOK // L
SCAP = 256

_PARAMS = pltpu.CompilerParams(use_tc_tiling_on_sc=True, needs_layout_passes=False)
_MESH = plsc.VectorSubcoreMesh(core_axis_name="c", subcore_axis_name="s")


def _wid():
    return lax.axis_index("s") * NC + lax.axis_index("c")


def _extract_body(x_hbm, tokT_hbm, tail_hbm, inter_hbm,
                  x_v, myv_v, myp_v, blk_v, tail_v, stag_v, idx_v, tmpv_v, tmpp_v, sem):
    wid = _wid()
    iota = lax.iota(jnp.int32, L)

    pltpu.sync_copy(x_hbm, x_v)

    def _fill(r, carry):
        for c in range(DP // L):
            idx_v[r, pl.ds(c * L, L)] = jnp.full((L,), -1, jnp.int32)
        return carry

    lax.fori_loop(0, NTOK // DP, _fill, 0)

    def _scan(i, cnt):
        v16 = x_v[i >> 7, pl.ds((i & 127) * L, L)]
        mask = lax.rem(lax.shift_right_logical(v16, 7), NW) == wid
        m32 = mask.astype(jnp.int32)
        slot = cnt + plsc.cumsum(m32) - 1
        plsc.store_scatter(myv_v, [slot], v16, mask=mask)
        plsc.store_scatter(myp_v, [slot], i * L + iota, mask=mask)
        return cnt + jnp.sum(m32)

    mycnt = lax.fori_loop(0, NCHUNK, _scan, 0)
    myrounds = lax.div(mycnt + (L - 1), L)

    def _blk(bi, ptr):
        j = wid + bi * NW
        is_tail = j == NFULL

        @pl.when(j < NFULL)
        def _fetch():
            pltpu.sync_copy(tokT_hbm.at[:, pl.ds(j * VBLK, VBLK)], blk_v)

        @pl.when(is_tail)
        def _fetch_tail():
            pltpu.sync_copy(tail_hbm, tail_v)

        def _round(r, ptr):
            mv16 = myv_v[pl.ds(r * L, L)]
            mp16 = myp_v[pl.ds(r * L, L)]
            live = (r * L + iota) < mycnt
            mask = (lax.shift_right_logical(mv16, 7) == j) & live
            m32 = mask.astype(jnp.int32)
            c = jnp.sum(m32)
            slot = plsc.cumsum(m32) - 1
            plsc.store_scatter(tmpv_v, [slot], lax.rem(mv16, VBLK), mask=mask)
            plsc.store_scatter(tmpp_v, [slot], mp16, mask=mask)

            @pl.when(c > 0)
            def _emit():
                vloc = tmpv_v[pl.ds(0, L)]
                pos = tmpp_v[pl.ds(0, L)]
                emit_mask = iota < c
                row = ptr + iota
                plsc.store_scatter(
                    idx_v,
                    [lax.shift_right_logical(row, 7), lax.rem(row, DP)],
                    pos,
                    mask=emit_mask,
                )
                srow = lax.rem(row, SCAP)

                def _dcol(k, carry):
                    dd = lax.rem(iota + k, L)
                    for cc in range(D // L):
                        dvec = dd + cc * L

                        @pl.when(jnp.logical_not(is_tail))
                        def _from_blk():
                            val = plsc.load_gather(blk_v, [dvec, vloc], mask=emit_mask)
                            plsc.store_scatter(stag_v, [srow, dvec], val, mask=emit_mask)

                        @pl.when(is_tail)
                        def _from_tail():
                            val = plsc.load_gather(tail_v, [vloc, dvec], mask=emit_mask)
                            plsc.store_scatter(stag_v, [srow, dvec], val, mask=emit_mask)

                    return carry

                lax.fori_loop(0, L, _dcol, 0)

            new_ptr = ptr + c

            @pl.when(lax.shift_right_logical(new_ptr, 7) > lax.shift_right_logical(ptr, 7))
            def _flush():
                cid = lax.shift_right_logical(ptr, 7)
                half = lax.rem(cid, 2) * (SCAP // 2)
                pltpu.async_copy(
                    stag_v.at[pl.ds(half, SCAP // 2)],
                    inter_hbm.at[plsc.Indices(idx_v.at[cid], ignored_value=-1)],
                    sem,
                ).wait()

            return new_ptr

        return lax.fori_loop(0, myrounds, _round, ptr)

    ptr = lax.fori_loop(0, BLK_PER_W, _blk, 0)

    @pl.when(lax.rem(ptr, DP) > 0)
    def _last():
        cid = lax.shift_right_logical(ptr, 7)
        half = lax.rem(cid, 2) * (SCAP // 2)
        pltpu.async_copy(
            stag_v.at[pl.ds(half, SCAP // 2)],
            inter_hbm.at[plsc.Indices(idx_v.at[cid], ignored_value=-1)],
            sem,
        ).wait()


def _posadd_body(inter_hbm, pos_hbm, out_hbm, in_v, pos_v, out_v, sem):
    wid = _wid()
    b = wid // SEQ_W
    s0 = (wid % SEQ_W) * B_W
    iota = lax.iota(jnp.int32, L)
    pltpu.sync_copy(inter_hbm.at[pl.ds(wid * B_W, B_W)], in_v)
    pltpu.sync_copy(pos_hbm.at[:, pl.ds(s0, B_W)], pos_v)

    def _grp(g, carry):
        t16 = g * L + iota

        def _k(k, carry2):
            dd = lax.rem(iota + k, L)
            for cc in range(D // L):
                dvec = dd + cc * L
                val = plsc.load_gather(in_v, [t16, dvec])
                pval = plsc.load_gather(pos_v, [dvec, t16])
                plsc.store_scatter(out_v, [dvec, t16], val + pval)
            return carry2

        return lax.fori_loop(0, L, _k, carry)

    lax.fori_loop(0, B_W // L, _grp, 0)
    pltpu.sync_copy(out_v, out_hbm.at[b, :, pl.ds(s0, B_W)])


def kernel(x, token_table, pos_table):
    tokT = token_table.T
    posT = pos_table.T
    tail_pad = jnp.pad(token_table[NFULL * VBLK :, :], ((0, 0), (0, DP - D)))
    inter = pl.kernel(
        _extract_body,
        mesh=_MESH,
        out_type=jax.ShapeDtypeStruct((NTOK, DP), jnp.float32),
        scratch_types=[
            pltpu.VMEM((BATCH, SEQ), jnp.int32),
            pltpu.VMEM((NTOK,), jnp.int32),
            pltpu.VMEM((NTOK,), jnp.int32),
            pltpu.VMEM((D, VBLK), jnp.float32),
            pltpu.VMEM((VREM, DP), jnp.float32),
            pltpu.VMEM((SCAP, DP), jnp.float32),
            pltpu.VMEM((NTOK // DP, DP), jnp.int32),
            pltpu.VMEM((L,), jnp.int32),
            pltpu.VMEM((L,), jnp.int32),
            pltpu.SemaphoreType.DMA,
        ],
        compiler_params=_PARAMS,
    )(x, tokT, tail_pad)
    outT = pl.kernel(
        _posadd_body,
        mesh=_MESH,
        out_type=jax.ShapeDtypeStruct((BATCH, D, SEQ), jnp.float32),
        scratch_types=[
            pltpu.VMEM((B_W, DP), jnp.float32),
            pltpu.VMEM((D, B_W), jnp.float32),
            pltpu.VMEM((D, B_W), jnp.float32),
            pltpu.SemaphoreType.DMA,
        ],
        compiler_params=_PARAMS,
    )(inter, posT)
    return outT.transpose(0, 2, 1)


# speedup vs baseline: 2.5871x; 1.9636x over previous
import jax
import jax.numpy as jnp
from jax import lax
from jax.experimental import pallas as pl
from jax.experimental.pallas import tpu as pltpu
from jax.experimental.pallas import tpu_sc as plsc

BATCH = 4
SEQ = 2048
D = 64
NTOK = BATCH * SEQ

NC = 2
NS = 16
L = 16
NW = NC * NS
B_W = NTOK // NW
SEQ_W = SEQ // B_W
CHUNK = 128

_PARAMS = pltpu.CompilerParams(use_tc_tiling_on_sc=False, needs_layout_passes=False)
_MESH = plsc.VectorSubcoreMesh(core_axis_name="c", subcore_axis_name="s")


def _body(x_hbm, tok_hbm, pos_hbm, out_hbm, idx_v, rows_v, pos_v, out_v, sem):
    wid = lax.axis_index("s") * NC + lax.axis_index("c")
    b = wid // SEQ_W
    s0 = (wid % SEQ_W) * B_W
    iota = lax.iota(jnp.int32, L)
    pltpu.sync_copy(x_hbm.at[b, pl.ds(s0, B_W)], idx_v)
    copies = [
        pltpu.async_copy(
            tok_hbm.at[idx_v.at[pl.ds(c * CHUNK, CHUNK)]],
            rows_v.at[pl.ds(c * CHUNK, CHUNK)],
            sem,
        )
        for c in range(B_W // CHUNK)
    ]
    pltpu.sync_copy(pos_hbm.at[:, pl.ds(s0, B_W)], pos_v)
    for cp in copies:
        cp.wait()

    def _grp(g, carry):
        t16 = g * L + iota

        def _k(k, carry2):
            dd = lax.rem(iota + k, L)
            for cc in range(D // L):
                dvec = dd + cc * L
                val = plsc.load_gather(rows_v, [t16, dvec])
                pval = plsc.load_gather(pos_v, [dvec, t16])
                plsc.store_scatter(out_v, [dvec, t16], val + pval)
            return carry2

        return lax.fori_loop(0, L, _k, carry)

    lax.fori_loop(0, B_W // L, _grp, 0)
    pltpu.sync_copy(out_v, out_hbm.at[b, :, pl.ds(s0, B_W)])


def kernel(x, token_table, pos_table):
    posT = pos_table.T
    outT = pl.kernel(
        _body,
        mesh=_MESH,
        out_type=jax.ShapeDtypeStruct((BATCH, D, SEQ), jnp.float32),
        scratch_types=[
            pltpu.VMEM((B_W,), jnp.int32),
            pltpu.VMEM((B_W, D), jnp.float32),
            pltpu.VMEM((D, B_W), jnp.float32),
            pltpu.VMEM((D, B_W), jnp.float32),
            pltpu.SemaphoreType.DMA,
        ],
        compiler_params=_PARAMS,
    )(x, token_table, posT)
    return outT.transpose(0, 2, 1)


# speedup vs baseline: 2.7057x; 1.0459x over previous
import jax
import jax.numpy as jnp
from jax import lax
from jax.experimental import pallas as pl
from jax.experimental.pallas import tpu as pltpu
from jax.experimental.pallas import tpu_sc as plsc

BATCH = 4
SEQ = 2048
D = 64
DP = 128
VOCAB = 100000
NTOK = BATCH * SEQ

NC = 2
NS = 16
L = 16
NW = NC * NS
B_W = NTOK // NW
SEQ_W = SEQ // B_W

VBLK = 128
NFULL = VOCAB // VBLK
VREM = VOCAB - NFULL * VBLK
NBLK = NFULL + 1
BLK_PER_W = -(-NBLK // NW)
NCHUNK = NTOK // L
SCAP = 256

_PARAMS = pltpu.CompilerParams(use_tc_tiling_on_sc=True, needs_layout_passes=False)
_MESH = plsc.VectorSubcoreMesh(core_axis_name="c", subcore_axis_name="s")


def _wid():
    return lax.axis_index("s") * NC + lax.axis_index("c")


def _extract_body(x_hbm, tokT_hbm, tail_hbm, inter_hbm,
                  x_v, myv_v, myp_v, blk_v, tail_v, stag_v, idx_v, pndv_v, pndp_v, sem):
    wid = _wid()
    iota = lax.iota(jnp.int32, L)

    pltpu.sync_copy(x_hbm, x_v)

    def _fill(r, carry):
        for c in range(DP // L):
            idx_v[r, pl.ds(c * L, L)] = jnp.full((L,), -1, jnp.int32)
        return carry

    lax.fori_loop(0, NTOK // DP, _fill, 0)

    def _count_of(csum):
        return jnp.squeeze(lax.slice(csum, (L - 1,), (L,)))

    def _scan(i, cnt):
        v16 = x_v[i >> 7, pl.ds((i & 127) * L, L)]
        mask = lax.rem(lax.shift_right_logical(v16, 7), NW) == wid
        csum = plsc.cumsum(mask.astype(jnp.int32))
        slot = cnt + csum - 1
        plsc.store_scatter(myv_v, [slot], v16, mask=mask)
        plsc.store_scatter(myp_v, [slot], i * L + iota, mask=mask)
        return cnt + _count_of(csum)

    mycnt = lax.fori_loop(0, NCHUNK, _scan, 0)
    myrounds = lax.div(mycnt + (L - 1), L)

    def _blk(bi, ptr):
        j = wid + bi * NW
        is_tail = j == NFULL

        @pl.when(j < NFULL)
        def _fetch():
            pltpu.sync_copy(tokT_hbm.at[:, pl.ds(j * VBLK, VBLK)], blk_v)

        @pl.when(is_tail)
        def _fetch_tail():
            pltpu.sync_copy(tail_hbm, tail_v)

        def _collect(r, pcnt):
            mv16 = myv_v[pl.ds(r * L, L)]
            mp16 = myp_v[pl.ds(r * L, L)]
            live = (r * L + iota) < mycnt
            mask = (lax.shift_right_logical(mv16, 7) == j) & live
            csum = plsc.cumsum(mask.astype(jnp.int32))
            slot = pcnt + csum - 1
            plsc.store_scatter(pndv_v, [slot], lax.rem(mv16, VBLK), mask=mask)
            plsc.store_scatter(pndp_v, [slot], mp16, mask=mask)
            return pcnt + _count_of(csum)

        pcnt = lax.fori_loop(0, myrounds, _collect, 0)
        erounds = lax.div(pcnt + (L - 1), L)

        def _emit(e, ptr):
            vloc = pndv_v[pl.ds(e * L, L)]
            pos = pndp_v[pl.ds(e * L, L)]
            emit_mask = (e * L + iota) < pcnt
            c = jnp.minimum(pcnt - e * L, L)
            row = ptr + iota
            plsc.store_scatter(
                idx_v,
                [lax.shift_right_logical(row, 7), lax.rem(row, DP)],
                pos,
                mask=emit_mask,
            )
            srow = lax.rem(row, SCAP)

            def _dcol(k, carry):
                dd = lax.rem(iota + k, L)
                for cc in range(D // L):
                    dvec = dd + cc * L

                    @pl.when(jnp.logical_not(is_tail))
                    def _from_blk():
                        val = plsc.load_gather(blk_v, [dvec, vloc], mask=emit_mask)
                        plsc.store_scatter(stag_v, [srow, dvec], val, mask=emit_mask)

                    @pl.when(is_tail)
                    def _from_tail():
                        val = plsc.load_gather(tail_v, [vloc, dvec], mask=emit_mask)
                        plsc.store_scatter(stag_v, [srow, dvec], val, mask=emit_mask)

                return carry

            lax.fori_loop(0, L, _dcol, 0)
            new_ptr = ptr + c

            @pl.when(lax.shift_right_logical(new_ptr, 7) > lax.shift_right_logical(ptr, 7))
            def _flush():
                cid = lax.shift_right_logical(ptr, 7)
                half = lax.rem(cid, 2) * (SCAP // 2)
                pltpu.async_copy(
                    stag_v.at[pl.ds(half, SCAP // 2)],
                    inter_hbm.at[plsc.Indices(idx_v.at[cid], ignored_value=-1)],
                    sem,
                ).wait()

            return new_ptr

        return lax.fori_loop(0, erounds, _emit, ptr)

    ptr = lax.fori_loop(0, BLK_PER_W, _blk, 0)

    @pl.when(lax.rem(ptr, DP) > 0)
    def _last():
        cid = lax.shift_right_logical(ptr, 7)
        half = lax.rem(cid, 2) * (SCAP // 2)
        pltpu.async_copy(
            stag_v.at[pl.ds(half, SCAP // 2)],
            inter_hbm.at[plsc.Indices(idx_v.at[cid], ignored_value=-1)],
            sem,
        ).wait()


def _posadd_body(inter_hbm, pos_hbm, out_hbm, in_v, pos_v, out_v, sem):
    wid = _wid()
    b = wid // SEQ_W
    s0 = (wid % SEQ_W) * B_W
    iota = lax.iota(jnp.int32, L)
    pltpu.sync_copy(inter_hbm.at[pl.ds(wid * B_W, B_W)], in_v)
    pltpu.sync_copy(pos_hbm.at[:, pl.ds(s0, B_W)], pos_v)

    def _grp(g, carry):
        t16 = g * L + iota

        def _k(k, carry2):
            dd = lax.rem(iota + k, L)
            for cc in range(D // L):
                dvec = dd + cc * L
                val = plsc.load_gather(in_v, [t16, dvec])
                pval = plsc.load_gather(pos_v, [dvec, t16])
                plsc.store_scatter(out_v, [dvec, t16], val + pval)
            return carry2

        return lax.fori_loop(0, L, _k, carry)

    lax.fori_loop(0, B_W // L, _grp, 0)
    pltpu.sync_copy(out_v, out_hbm.at[b, :, pl.ds(s0, B_W)])


def kernel(x, token_table, pos_table):
    tokT = token_table.T
    posT = pos_table.T
    tail_pad = jnp.pad(token_table[NFULL * VBLK :, :], ((0, 0), (0, DP - D)))
    inter = pl.kernel(
        _extract_body,
        mesh=_MESH,
        out_type=jax.ShapeDtypeStruct((NTOK, DP), jnp.float32),
        scratch_types=[
            pltpu.VMEM((BATCH, SEQ), jnp.int32),
            pltpu.VMEM((NTOK,), jnp.int32),
            pltpu.VMEM((NTOK,), jnp.int32),
            pltpu.VMEM((D, VBLK), jnp.float32),
            pltpu.VMEM((VREM, DP), jnp.float32),
            pltpu.VMEM((SCAP, DP), jnp.float32),
            pltpu.VMEM((NTOK // DP, DP), jnp.int32),
            pltpu.VMEM((NTOK + L,), jnp.int32),
            pltpu.VMEM((NTOK + L,), jnp.int32),
            pltpu.SemaphoreType.DMA,
        ],
        compiler_params=_PARAMS,
    )(x, tokT, tail_pad)
    outT = pl.kernel(
        _posadd_body,
        mesh=_MESH,
        out_type=jax.ShapeDtypeStruct((BATCH, D, SEQ), jnp.float32),
        scratch_types=[
            pltpu.VMEM((B_W, DP), jnp.float32),
            pltpu.VMEM((D, B_W), jnp.float32),
            pltpu.VMEM((D, B_W), jnp.float32),
            pltpu.SemaphoreType.DMA,
        ],
        compiler_params=_PARAMS,
    )(inter, posT)
    return outT.transpose(0, 2, 1)


# speedup vs baseline: 3.3711x; 1.2459x over previous
import jax
import jax.numpy as jnp
from jax import lax
from jax.experimental import pallas as pl
from jax.experimental.pallas import tpu as pltpu
from jax.experimental.pallas import tpu_sc as plsc

BATCH = 4
SEQ = 2048
D = 64
DP = 128
VOCAB = 100000
NTOK = BATCH * SEQ

NC = 2
NS = 16
L = 16
NW = NC * NS
B_W = NTOK // NW
SEQ_W = SEQ // B_W

VBLK = 128
NFULL = VOCAB // VBLK
VREM = VOCAB - NFULL * VBLK
NBLK = NFULL + 1
BLK_PER_W = -(-NBLK // NW)
NCHUNK = NTOK // L
SCAP = 256

_PARAMS = pltpu.CompilerParams(use_tc_tiling_on_sc=True, needs_layout_passes=False)
_MESH = plsc.VectorSubcoreMesh(core_axis_name="c", subcore_axis_name="s")


def _wid():
    return lax.axis_index("s") * NC + lax.axis_index("c")


def _extract_body(x_hbm, tokT_hbm, tail_hbm, inter_hbm,
                  x_v, myv_v, myp_v, blk2_v, tail_v, stag_v, idx_v, pndv_v, pndp_v, sem, psem):
    wid = _wid()
    iota = lax.iota(jnp.int32, L)

    pltpu.sync_copy(x_hbm, x_v)

    def _fill(r, carry):
        for c in range(DP // L):
            idx_v[r, pl.ds(c * L, L)] = jnp.full((L,), -1, jnp.int32)
        return carry

    lax.fori_loop(0, NTOK // DP, _fill, 0)

    def _count_of(csum):
        return jnp.squeeze(lax.slice(csum, (L - 1,), (L,)))

    def _scan(i, cnt):
        v16 = x_v[i >> 7, pl.ds((i & 127) * L, L)]
        mask = lax.rem(lax.shift_right_logical(v16, 7), NW) == wid
        csum = plsc.cumsum(mask.astype(jnp.int32))
        slot = cnt + csum - 1
        plsc.store_scatter(myv_v, [slot], v16, mask=mask)
        plsc.store_scatter(myp_v, [slot], i * L + iota, mask=mask)
        return cnt + _count_of(csum)

    mycnt = lax.fori_loop(0, NCHUNK, _scan, 0)
    myrounds = lax.div(mycnt + (L - 1), L)

    pltpu.async_copy(tokT_hbm.at[:, pl.ds(wid * VBLK, VBLK)], blk2_v.at[0], psem)

    def _blk(bi, ptr):
        j = wid + bi * NW
        is_tail = j == NFULL
        par = lax.rem(bi, 2)
        blk_v = blk2_v.at[par]

        @pl.when(j < NFULL)
        def _fetch_wait():
            pltpu.make_async_copy(
                tokT_hbm.at[:, pl.ds(0, VBLK)], blk2_v.at[par], psem
            ).wait()

        @pl.when(j + NW < NFULL)
        def _prefetch_next():
            pltpu.async_copy(
                tokT_hbm.at[:, pl.ds((j + NW) * VBLK, VBLK)],
                blk2_v.at[lax.rem(bi + 1, 2)],
                psem,
            )

        @pl.when(is_tail)
        def _fetch_tail():
            pltpu.sync_copy(tail_hbm, tail_v)

        def _collect(r, pcnt):
            mv16 = myv_v[pl.ds(r * L, L)]
            mp16 = myp_v[pl.ds(r * L, L)]
            live = (r * L + iota) < mycnt
            mask = (lax.shift_right_logical(mv16, 7) == j) & live
            csum = plsc.cumsum(mask.astype(jnp.int32))
            slot = pcnt + csum - 1
            plsc.store_scatter(pndv_v, [slot], lax.rem(mv16, VBLK), mask=mask)
            plsc.store_scatter(pndp_v, [slot], mp16, mask=mask)
            return pcnt + _count_of(csum)

        pcnt = lax.fori_loop(0, myrounds, _collect, 0)
        erounds = lax.div(pcnt + (L - 1), L)

        def _emit(e, ptr):
            vloc = pndv_v[pl.ds(e * L, L)]
            pos = pndp_v[pl.ds(e * L, L)]
            emit_mask = (e * L + iota) < pcnt
            c = jnp.minimum(pcnt - e * L, L)
            row = ptr + iota
            plsc.store_scatter(
                idx_v,
                [lax.shift_right_logical(row, 7), lax.rem(row, DP)],
                pos,
                mask=emit_mask,
            )
            srow = lax.rem(row, SCAP)

            def _dcol(k, carry):
                dd = lax.rem(iota + k, L)
                for cc in range(D // L):
                    dvec = dd + cc * L

                    @pl.when(jnp.logical_not(is_tail))
                    def _from_blk():
                        val = plsc.load_gather(blk_v, [dvec, vloc], mask=emit_mask)
                        plsc.store_scatter(stag_v, [srow, dvec], val, mask=emit_mask)

                    @pl.when(is_tail)
                    def _from_tail():
                        val = plsc.load_gather(tail_v, [vloc, dvec], mask=emit_mask)
                        plsc.store_scatter(stag_v, [srow, dvec], val, mask=emit_mask)

                return carry

            lax.fori_loop(0, L, _dcol, 0)
            new_ptr = ptr + c

            @pl.when(lax.shift_right_logical(new_ptr, 7) > lax.shift_right_logical(ptr, 7))
            def _flush():
                cid = lax.shift_right_logical(ptr, 7)
                half = lax.rem(cid, 2) * (SCAP // 2)
                pltpu.async_copy(
                    stag_v.at[pl.ds(half, SCAP // 2)],
                    inter_hbm.at[plsc.Indices(idx_v.at[cid], ignored_value=-1)],
                    sem,
                ).wait()

            return new_ptr

        return lax.fori_loop(0, erounds, _emit, ptr)

    ptr = lax.fori_loop(0, BLK_PER_W, _blk, 0)

    @pl.when(lax.rem(ptr, DP) > 0)
    def _last():
        cid = lax.shift_right_logical(ptr, 7)
        half = lax.rem(cid, 2) * (SCAP // 2)
        pltpu.async_copy(
            stag_v.at[pl.ds(half, SCAP // 2)],
            inter_hbm.at[plsc.Indices(idx_v.at[cid], ignored_value=-1)],
            sem,
        ).wait()


def _posadd_body(inter_hbm, pos_hbm, out_hbm, in_v, pos_v, out_v, sem):
    wid = _wid()
    b = wid // SEQ_W
    s0 = (wid % SEQ_W) * B_W
    iota = lax.iota(jnp.int32, L)
    pltpu.sync_copy(inter_hbm.at[pl.ds(wid * B_W, B_W)], in_v)
    pltpu.sync_copy(pos_hbm.at[:, pl.ds(s0, B_W)], pos_v)

    def _grp(g, carry):
        t16 = g * L + iota

        def _k(k, carry2):
            dd = lax.rem(iota + k, L)
            for cc in range(D // L):
                dvec = dd + cc * L
                val = plsc.load_gather(in_v, [t16, dvec])
                pval = plsc.load_gather(pos_v, [dvec, t16])
                plsc.store_scatter(out_v, [dvec, t16], val + pval)
            return carry2

        return lax.fori_loop(0, L, _k, carry)

    lax.fori_loop(0, B_W // L, _grp, 0)
    pltpu.sync_copy(out_v, out_hbm.at[b, :, pl.ds(s0, B_W)])


def kernel(x, token_table, pos_table):
    tokT = token_table.T
    posT = pos_table.T
    tail_pad = jnp.pad(token_table[NFULL * VBLK :, :], ((0, 0), (0, DP - D)))
    inter = pl.kernel(
        _extract_body,
        mesh=_MESH,
        out_type=jax.ShapeDtypeStruct((NTOK, DP), jnp.float32),
        scratch_types=[
            pltpu.VMEM((BATCH, SEQ), jnp.int32),
            pltpu.VMEM((NTOK,), jnp.int32),
            pltpu.VMEM((NTOK,), jnp.int32),
            pltpu.VMEM((2, D, VBLK), jnp.float32),
            pltpu.VMEM((VREM, DP), jnp.float32),
            pltpu.VMEM((SCAP, DP), jnp.float32),
            pltpu.VMEM((NTOK // DP, DP), jnp.int32),
            pltpu.VMEM((NTOK + L,), jnp.int32),
            pltpu.VMEM((NTOK + L,), jnp.int32),
            pltpu.SemaphoreType.DMA,
            pltpu.SemaphoreType.DMA,
        ],
        compiler_params=_PARAMS,
    )(x, tokT, tail_pad)
    outT = pl.kernel(
        _posadd_body,
        mesh=_MESH,
        out_type=jax.ShapeDtypeStruct((BATCH, D, SEQ), jnp.float32),
        scratch_types=[
            pltpu.VMEM((B_W, DP), jnp.float32),
            pltpu.VMEM((D, B_W), jnp.float32),
            pltpu.VMEM((D, B_W), jnp.float32),
            pltpu.SemaphoreType.DMA,
        ],
        compiler_params=_PARAMS,
    )(inter, posT)
    return outT.transpose(0, 2, 1)
